# baseline (speedup 1.0000x reference)
import functools

import jax
import jax.numpy as jnp
from jax import lax
from jax.experimental import pallas as pl
from jax.experimental.pallas import tpu as pltpu
from jax.experimental.pallas import tpu_sc as plsc

B = 16384
EMB = 256
HID = 512
F32 = jnp.float32

NC, NS = 2, 16
NW = NC * NS
BPW = B // NW
CHUNK = 64


def _gelu(x):
    return 0.5 * x * (1.0 + lax.erf(x * 0.7071067811865476))


def _fold_body(ct, rt, lt, fW1, cW2, cb2, fb1, wc, wr, wl, m):
    bvec = jnp.dot(cb2[...].reshape(1, 64), fW1[512:576, :],
                   preferred_element_type=F32) + fb1[...].reshape(1, HID)
    wc[...] = jnp.dot(ct[...], fW1[0:256, :], preferred_element_type=F32) + bvec
    wr[...] = jnp.dot(rt[...], fW1[256:384, :], preferred_element_type=F32)
    wl[...] = jnp.dot(lt[...], fW1[384:512, :], preferred_element_type=F32)
    m[...] = jnp.dot(cW2[...], fW1[512:576, :], preferred_element_type=F32)


def _fold(ct, rt, lt, fW1, cW2, cb2, fb1):
    return pl.pallas_call(
        _fold_body,
        out_shape=(
            jax.ShapeDtypeStruct((200, HID), F32),
            jax.ShapeDtypeStruct((20, HID), F32),
            jax.ShapeDtypeStruct((100, HID), F32),
            jax.ShapeDtypeStruct((64, HID), F32),
        ),
    )(ct, rt, lt, fW1, cW2, cb2, fb1)


def _sc_gather_body(wc_hbm, wr_hbm, wl_hbm, cid_hbm, rid_hbm, lid_hbm,
                    out_hbm, cid_v, rid_v, lid_v, b0, b1, b2, sem):
    wid = lax.axis_index("s") * NC + lax.axis_index("c")
    base = wid * BPW
    pltpu.sync_copy(cid_hbm.at[pl.ds(base, BPW)], cid_v)
    pltpu.sync_copy(rid_hbm.at[pl.ds(base, BPW)], rid_v)
    pltpu.sync_copy(lid_hbm.at[pl.ds(base, BPW)], lid_v)

    def chunk(i, carry):
        o = i * CHUNK
        ca = pltpu.async_copy(wc_hbm.at[cid_v.at[pl.ds(o, CHUNK)]], b0, sem)
        cb = pltpu.async_copy(wr_hbm.at[rid_v.at[pl.ds(o, CHUNK)]], b1, sem)
        cc = pltpu.async_copy(wl_hbm.at[lid_v.at[pl.ds(o, CHUNK)]], b2, sem)
        ca.wait()
        cb.wait()
        cc.wait()

        def seg(k, carry2):
            r = k // 32
            j = (k % 32) * 16
            b0[r, pl.ds(j, 16)] = (b0[r, pl.ds(j, 16)] + b1[r, pl.ds(j, 16)]
                                   + b2[r, pl.ds(j, 16)])
            return carry2

        lax.fori_loop(0, CHUNK * 32, seg, 0)
        pltpu.sync_copy(b0, out_hbm.at[pl.ds(base + o, CHUNK)])
        return carry

    lax.fori_loop(0, BPW // CHUNK, chunk, 0)


def _sc_gather(wc, wr, wl, cid, rid, lid):
    mesh = plsc.VectorSubcoreMesh(core_axis_name="c", subcore_axis_name="s")
    return pl.kernel(
        _sc_gather_body,
        out_type=jax.ShapeDtypeStruct((B, HID), F32),
        mesh=mesh,
        scratch_types=[
            pltpu.VMEM((BPW,), jnp.int32),
            pltpu.VMEM((BPW,), jnp.int32),
            pltpu.VMEM((BPW,), jnp.int32),
            pltpu.VMEM((CHUNK, HID), F32),
            pltpu.VMEM((CHUNK, HID), F32),
            pltpu.VMEM((CHUNK, HID), F32),
            pltpu.SemaphoreType.DMA,
        ],
    )(wc, wr, wl, cid, rid, lid)


BLK = 1024


def _main_body(g, ch, cW1, cb1, m, fW2, fb2, o):
    h = _gelu(jnp.dot(ch[...], cW1[...], preferred_element_type=F32) + cb1[...])
    acc = g[...] + jnp.dot(h, m[...], preferred_element_type=F32)
    o[...] = jnp.dot(_gelu(acc), fW2[...], preferred_element_type=F32) + fb2[...]


def _tc_main(g, ch, cW1, cb1, m, fW2, fb2):
    grid = (B // BLK,)
    return pl.pallas_call(
        _main_body,
        grid=grid,
        in_specs=[
            pl.BlockSpec((BLK, HID), lambda i: (i, 0)),
            pl.BlockSpec((BLK, 16), lambda i: (i, 0)),
            pl.BlockSpec((16, 64), lambda i: (0, 0)),
            pl.BlockSpec((64,), lambda i: (0,)),
            pl.BlockSpec((64, HID), lambda i: (0, 0)),
            pl.BlockSpec((HID, HID), lambda i: (0, 0)),
            pl.BlockSpec((HID,), lambda i: (0,)),
        ],
        out_specs=pl.BlockSpec((BLK, HID), lambda i: (i, 0)),
        out_shape=jax.ShapeDtypeStruct((B, HID), F32),
    )(g, ch, cW1, cb1, m, fW2, fb2)


def kernel(country_ids, region_ids, language_ids, characteristics,
           country_table, region_table, lang_table,
           cW1, cb1, cW2, cb2, fW1, fb1, fW2, fb2):
    cid = country_ids.astype(jnp.int32)
    rid = region_ids.astype(jnp.int32)
    lid = language_ids.astype(jnp.int32)
    wc, wr, wl, m = _fold(country_table, region_table, lang_table,
                          fW1, cW2, cb2, fb1)
    g = _sc_gather(wc, wr, wl, cid, rid, lid)
    return _tc_main(g, characteristics, cW1, cb1, m, fW2, fb2)


# speedup vs baseline: 1.8385x; 1.8385x over previous
import jax
import jax.numpy as jnp
from jax import lax
from jax.experimental import pallas as pl
from jax.experimental.pallas import tpu as pltpu
from jax.experimental.pallas import tpu_sc as plsc

B = 16384
EMB = 256
HID = 512
F32 = jnp.float32

NC, NS = 2, 16
NW = NC * NS
BPW = B // NW
CHUNK = 32
NCHUNK = BPW // CHUNK


def _gelu(x):
    return 0.5 * x * (1.0 + lax.erf(x * 0.7071067811865476))


def _fold_body(ct, rt, lt, fW1, cW2, cb2, fb1, wc, wrl, m):
    bvec = jnp.dot(cb2[...].reshape(1, 64), fW1[512:576, :],
                   preferred_element_type=F32) + fb1[...].reshape(1, HID)
    wc[...] = jnp.dot(ct[...], fW1[0:256, :], preferred_element_type=F32) + bvec
    wr = jnp.dot(rt[...], fW1[256:384, :], preferred_element_type=F32)
    wl = jnp.dot(lt[...], fW1[384:512, :], preferred_element_type=F32)
    wrl[...] = (wr.reshape(20, 1, HID) + wl.reshape(1, 100, HID)).reshape(
        2000, HID)
    m[...] = jnp.dot(cW2[...], fW1[512:576, :], preferred_element_type=F32)


def _fold(ct, rt, lt, fW1, cW2, cb2, fb1):
    return pl.pallas_call(
        _fold_body,
        out_shape=(
            jax.ShapeDtypeStruct((200, HID), F32),
            jax.ShapeDtypeStruct((2000, HID), F32),
            jax.ShapeDtypeStruct((64, HID), F32),
        ),
    )(ct, rt, lt, fW1, cW2, cb2, fb1)


def _sc_gather_body(wc_hbm, wrl_hbm, cid_hbm, pid_hbm, gc_hbm, grl_hbm,
                    cid_v, pid_v, bc0, bc1, br0, br1,
                    sc0, sc1, sr0, sr1):
    wid = lax.axis_index("s") * NC + lax.axis_index("c")
    base = wid * BPW
    pltpu.sync_copy(cid_hbm.at[pl.ds(base, BPW)], cid_v)
    pltpu.sync_copy(pid_hbm.at[pl.ds(base, BPW)], pid_v)

    bcs, brs = (bc0, bc1), (br0, br1)
    scs, srs = (sc0, sc1), (sr0, sr1)
    gdesc = [None] * NCHUNK

    for i in range(NCHUNK):
        o = i * CHUNK
        p = i % 2
        gdesc[i] = (
            pltpu.async_copy(wc_hbm.at[cid_v.at[pl.ds(o, CHUNK)]],
                             bcs[p], scs[p]),
            pltpu.async_copy(wrl_hbm.at[pid_v.at[pl.ds(o, CHUNK)]],
                             brs[p], srs[p]),
        )
        if i >= 1:
            q = (i - 1) % 2
            oo = (i - 1) * CHUNK
            gdesc[i - 1][0].wait()
            pltpu.sync_copy(bcs[q], gc_hbm.at[pl.ds(base + oo, CHUNK)])
            gdesc[i - 1][1].wait()
            pltpu.sync_copy(brs[q], grl_hbm.at[pl.ds(base + oo, CHUNK)])
    q = (NCHUNK - 1) % 2
    oo = (NCHUNK - 1) * CHUNK
    gdesc[NCHUNK - 1][0].wait()
    pltpu.sync_copy(bcs[q], gc_hbm.at[pl.ds(base + oo, CHUNK)])
    gdesc[NCHUNK - 1][1].wait()
    pltpu.sync_copy(brs[q], grl_hbm.at[pl.ds(base + oo, CHUNK)])


def _sc_gather(wc, wrl, cid, pid):
    mesh = plsc.VectorSubcoreMesh(core_axis_name="c", subcore_axis_name="s")
    return pl.kernel(
        _sc_gather_body,
        out_type=(jax.ShapeDtypeStruct((B, HID), F32),
                  jax.ShapeDtypeStruct((B, HID), F32)),
        mesh=mesh,
        scratch_types=[
            pltpu.VMEM((BPW,), jnp.int32),
            pltpu.VMEM((BPW,), jnp.int32),
            pltpu.VMEM((CHUNK, HID), F32),
            pltpu.VMEM((CHUNK, HID), F32),
            pltpu.VMEM((CHUNK, HID), F32),
            pltpu.VMEM((CHUNK, HID), F32),
            pltpu.SemaphoreType.DMA,
            pltpu.SemaphoreType.DMA,
            pltpu.SemaphoreType.DMA,
            pltpu.SemaphoreType.DMA,
        ],
    )(wc, wrl, cid, pid)


BLK = 1024


def _main_body(gc, grl, ch, cW1, cb1, m, fW2, fb2, o):
    h = _gelu(jnp.dot(ch[...], cW1[...], preferred_element_type=F32) + cb1[...])
    acc = gc[...] + grl[...] + jnp.dot(h, m[...], preferred_element_type=F32)
    o[...] = jnp.dot(_gelu(acc), fW2[...], preferred_element_type=F32) + fb2[...]


def _tc_main(gc, grl, ch, cW1, cb1, m, fW2, fb2):
    grid = (B // BLK,)
    return pl.pallas_call(
        _main_body,
        grid=grid,
        in_specs=[
            pl.BlockSpec((BLK, HID), lambda i: (i, 0)),
            pl.BlockSpec((BLK, HID), lambda i: (i, 0)),
            pl.BlockSpec((BLK, 16), lambda i: (i, 0)),
            pl.BlockSpec((16, 64), lambda i: (0, 0)),
            pl.BlockSpec((64,), lambda i: (0,)),
            pl.BlockSpec((64, HID), lambda i: (0, 0)),
            pl.BlockSpec((HID, HID), lambda i: (0, 0)),
            pl.BlockSpec((HID,), lambda i: (0,)),
        ],
        out_specs=pl.BlockSpec((BLK, HID), lambda i: (i, 0)),
        out_shape=jax.ShapeDtypeStruct((B, HID), F32),
    )(gc, grl, ch, cW1, cb1, m, fW2, fb2)


def kernel(country_ids, region_ids, language_ids, characteristics,
           country_table, region_table, lang_table,
           cW1, cb1, cW2, cb2, fW1, fb1, fW2, fb2):
    cid = country_ids.astype(jnp.int32)
    pid = region_ids.astype(jnp.int32) * 100 + language_ids.astype(jnp.int32)
    wc, wrl, m = _fold(country_table, region_table, lang_table,
                       fW1, cW2, cb2, fb1)
    gc, grl = _sc_gather(wc, wrl, cid, pid)
    return _tc_main(gc, grl, characteristics, cW1, cb1, m, fW2, fb2)
